# baseline (speedup 1.0000x reference)
import functools

import jax
import jax.numpy as jnp
from jax.experimental import pallas as pl
from jax.experimental.pallas import tpu as pltpu

INPUT_DIM = 256
EMB_DIM = 128
HIDDEN_DIM = 512
BATCH = 128
SEQ = 200
H4 = 4 * HIDDEN_DIM


def _lstm_kernel(urls_ref, embt_ref, wihf_ref, whhf_ref, bf_ref,
                 wihb_ref, whhb_ref, bb_ref,
                 fcw_ref, fcb_ref, fc1w_ref, fc1b_ref,
                 fc2w_ref, fc2b_ref, fc3w_ref, fc3b_ref,
                 out_ref, aux_ref,
                 mtf_scr, mtb_scr, hf_scr, cf_scr, hb_scr, cb_scr):
    f32 = jnp.float32
    bf16 = jnp.bfloat16

    embt = embt_ref[...]
    mtf_scr[...] = (jnp.dot(wihf_ref[...], embt, preferred_element_type=f32)
                    + bf_ref[...]).astype(bf16)
    mtb_scr[...] = (jnp.dot(wihb_ref[...], embt, preferred_element_type=f32)
                    + bb_ref[...]).astype(bf16)

    hf_scr[...] = jnp.zeros((HIDDEN_DIM, BATCH), bf16)
    hb_scr[...] = jnp.zeros((HIDDEN_DIM, BATCH), bf16)
    cf_scr[...] = jnp.zeros((HIDDEN_DIM, BATCH), f32)
    cb_scr[...] = jnp.zeros((HIDDEN_DIM, BATCH), f32)

    whhf = whhf_ref[...]
    whhb = whhb_ref[...]
    mtf = mtf_scr[...]
    mtb = mtb_scr[...]

    iota = jax.lax.broadcasted_iota(jnp.int32, (INPUT_DIM, BATCH), 0)

    def sig(x):
        return 0.5 * jnp.tanh(0.5 * x) + 0.5

    def step_dir(ids, h, c, mt, whh):
        onehot_t = (ids == iota).astype(bf16)
        gates = (jnp.dot(mt, onehot_t, preferred_element_type=f32)
                 + jnp.dot(whh, h, preferred_element_type=f32))
        i = sig(gates[0:HIDDEN_DIM, :])
        f = sig(gates[HIDDEN_DIM:2 * HIDDEN_DIM, :])
        g = jnp.tanh(gates[2 * HIDDEN_DIM:3 * HIDDEN_DIM, :])
        o = sig(gates[3 * HIDDEN_DIM:, :])
        c = f * c + i * g
        h = (o * jnp.tanh(c)).astype(bf16)
        return h, c

    def body(t, _):
        ids_f = urls_ref[t]
        ids_b = urls_ref[SEQ - 1 - t]
        hf, cf = step_dir(ids_f, hf_scr[...], cf_scr[...], mtf, whhf)
        hb, cb = step_dir(ids_b, hb_scr[...], cb_scr[...], mtb, whhb)
        hf_scr[...] = hf
        cf_scr[...] = cf
        hb_scr[...] = hb
        cb_scr[...] = cb
        return 0

    jax.lax.fori_loop(0, SEQ, body, 0)

    hiddenT = jnp.concatenate([hf_scr[...], hb_scr[...]], axis=0).astype(f32)
    aux_ref[...] = jnp.dot(fcw_ref[...], hiddenT, preferred_element_type=f32) + fcb_ref[...]
    o1 = jnp.dot(fc1w_ref[...], hiddenT, preferred_element_type=f32) + fc1b_ref[...]
    o2 = jnp.dot(fc2w_ref[...], o1, preferred_element_type=f32) + fc2b_ref[...]
    out_ref[...] = jnp.dot(fc3w_ref[...], o2, preferred_element_type=f32) + fc3b_ref[...]


@functools.partial(jax.jit, static_argnames=("interpret",))
def _run(urls, emb_table, W_ih_f, W_hh_f, b_f, W_ih_b, W_hh_b, b_b,
         fc_w, fc_b, fc1_w, fc1_b, fc2_w, fc2_b, fc3_w, fc3_b,
         interpret=False):
    urls3 = urls.T.reshape(SEQ, 1, BATCH).astype(jnp.int32)
    f32 = jnp.float32
    bf16 = jnp.bfloat16
    args = (
        urls3,
        emb_table.T,
        W_ih_f.astype(bf16), W_hh_f.astype(bf16), b_f.reshape(H4, 1),
        W_ih_b.astype(bf16), W_hh_b.astype(bf16), b_b.reshape(H4, 1),
        fc_w, fc_b.reshape(1, 1),
        fc1_w, fc1_b.reshape(H4, 1),
        fc2_w, fc2_b.reshape(2 * HIDDEN_DIM, 1),
        fc3_w, fc3_b.reshape(2, 1),
    )
    outT, auxT = pl.pallas_call(
        _lstm_kernel,
        out_shape=(
            jax.ShapeDtypeStruct((2, BATCH), f32),
            jax.ShapeDtypeStruct((1, BATCH), f32),
        ),
        scratch_shapes=[
            pltpu.VMEM((H4, INPUT_DIM), bf16),
            pltpu.VMEM((H4, INPUT_DIM), bf16),
            pltpu.VMEM((HIDDEN_DIM, BATCH), bf16),
            pltpu.VMEM((HIDDEN_DIM, BATCH), f32),
            pltpu.VMEM((HIDDEN_DIM, BATCH), bf16),
            pltpu.VMEM((HIDDEN_DIM, BATCH), f32),
        ],
        interpret=interpret,
    )(*args)
    return outT.T, auxT[0]


def kernel(urls, emb_table, W_ih_f, W_hh_f, b_f, W_ih_b, W_hh_b, b_b,
           fc_w, fc_b, fc1_w, fc1_b, fc2_w, fc2_b, fc3_w, fc3_b):
    return _run(urls, emb_table, W_ih_f, W_hh_f, b_f, W_ih_b, W_hh_b, b_b,
                fc_w, fc_b, fc1_w, fc1_b, fc2_w, fc2_b, fc3_w, fc3_b)


# speedup vs baseline: 1.2635x; 1.2635x over previous
import functools

import jax
import jax.numpy as jnp
from jax.experimental import pallas as pl
from jax.experimental.pallas import tpu as pltpu

INPUT_DIM = 256
EMB_DIM = 128
HIDDEN_DIM = 512
BATCH = 128
SEQ = 200
H4 = 4 * HIDDEN_DIM
CHUNK = 10


def _lstm_kernel(urls_ref, emb_ref, wihf_ref, whhf_ref, bf_ref,
                 wihb_ref, whhb_ref, bb_ref,
                 fcw_ref, fcb_ref, fc1w_ref, fc1b_ref,
                 fc2w_ref, fc2b_ref, fc3w_ref, fc3b_ref,
                 out_ref, aux_ref,
                 mf_scr, mb_scr, gif_scr, gib_scr,
                 hf_scr, cf_scr, hb_scr, cb_scr):
    f32 = jnp.float32
    bf16 = jnp.bfloat16
    rows = CHUNK * BATCH

    emb = emb_ref[...]
    mf_scr[...] = (jnp.dot(emb, wihf_ref[...], preferred_element_type=f32)
                   + bf_ref[...]).astype(bf16)
    mb_scr[...] = (jnp.dot(emb, wihb_ref[...], preferred_element_type=f32)
                   + bb_ref[...]).astype(bf16)

    hf_scr[...] = jnp.zeros((BATCH, HIDDEN_DIM), bf16)
    hb_scr[...] = jnp.zeros((BATCH, HIDDEN_DIM), bf16)
    cf_scr[...] = jnp.zeros((BATCH, HIDDEN_DIM), f32)
    cb_scr[...] = jnp.zeros((BATCH, HIDDEN_DIM), f32)

    iota = jax.lax.broadcasted_iota(jnp.int32, (rows, INPUT_DIM), 1)

    whhf = whhf_ref[...]
    whhb = whhb_ref[...]
    mf = mf_scr[...]
    mb = mb_scr[...]

    def sig(x):
        return 0.5 * jnp.tanh(0.5 * x) + 0.5

    def step_dir(gin, h, c, whh):
        gates = jnp.dot(h, whh, preferred_element_type=f32) + gin.astype(f32)
        i = sig(gates[:, 0:HIDDEN_DIM])
        f = sig(gates[:, HIDDEN_DIM:2 * HIDDEN_DIM])
        g = jnp.tanh(gates[:, 2 * HIDDEN_DIM:3 * HIDDEN_DIM])
        o = sig(gates[:, 3 * HIDDEN_DIM:])
        c = f * c + i * g
        h = (o * jnp.tanh(c)).astype(bf16)
        return h, c

    def chunk_body(k, _):
        ids_f = urls_ref[pl.ds(k * rows, rows), :]
        ids_b = urls_ref[pl.ds((SEQ * BATCH) - (k + 1) * rows, rows), :]
        gif_scr[...] = jnp.dot((ids_f == iota).astype(bf16), mf,
                               preferred_element_type=f32).astype(bf16)
        gib_scr[...] = jnp.dot((ids_b == iota).astype(bf16), mb,
                               preferred_element_type=f32).astype(bf16)

        def body(j, _):
            gf = gif_scr[pl.ds(j * BATCH, BATCH), :]
            gb = gib_scr[pl.ds((CHUNK - 1 - j) * BATCH, BATCH), :]
            hf, cf = step_dir(gf, hf_scr[...], cf_scr[...], whhf)
            hb, cb = step_dir(gb, hb_scr[...], cb_scr[...], whhb)
            hf_scr[...] = hf
            cf_scr[...] = cf
            hb_scr[...] = hb
            cb_scr[...] = cb
            return 0

        jax.lax.fori_loop(0, CHUNK, body, 0)
        return 0

    jax.lax.fori_loop(0, SEQ // CHUNK, chunk_body, 0)

    hidden = jnp.concatenate([hf_scr[...], hb_scr[...]], axis=1).astype(f32)
    aux_ref[...] = jnp.dot(hidden, fcw_ref[...], preferred_element_type=f32) + fcb_ref[...]
    o1 = jnp.dot(hidden, fc1w_ref[...], preferred_element_type=f32) + fc1b_ref[...]
    o2 = jnp.dot(o1, fc2w_ref[...], preferred_element_type=f32) + fc2b_ref[...]
    out_ref[...] = jnp.dot(o2, fc3w_ref[...], preferred_element_type=f32) + fc3b_ref[...]


@functools.partial(jax.jit, static_argnames=("interpret",))
def _run(urls, emb_table, W_ih_f, W_hh_f, b_f, W_ih_b, W_hh_b, b_b,
         fc_w, fc_b, fc1_w, fc1_b, fc2_w, fc2_b, fc3_w, fc3_b,
         interpret=False):
    urls_flat = urls.T.reshape(SEQ * BATCH, 1).astype(jnp.int32)
    f32 = jnp.float32
    bf16 = jnp.bfloat16
    args = (
        urls_flat,
        emb_table,
        W_ih_f.T, W_hh_f.T.astype(bf16), b_f.reshape(1, H4),
        W_ih_b.T, W_hh_b.T.astype(bf16), b_b.reshape(1, H4),
        fc_w.T, fc_b.reshape(1, 1),
        fc1_w.T, fc1_b.reshape(1, H4),
        fc2_w.T, fc2_b.reshape(1, 2 * HIDDEN_DIM),
        fc3_w.T, fc3_b.reshape(1, 2),
    )
    out, aux = pl.pallas_call(
        _lstm_kernel,
        out_shape=(
            jax.ShapeDtypeStruct((BATCH, 2), f32),
            jax.ShapeDtypeStruct((BATCH, 1), f32),
        ),
        scratch_shapes=[
            pltpu.VMEM((INPUT_DIM, H4), bf16),
            pltpu.VMEM((INPUT_DIM, H4), bf16),
            pltpu.VMEM((CHUNK * BATCH, H4), bf16),
            pltpu.VMEM((CHUNK * BATCH, H4), bf16),
            pltpu.VMEM((BATCH, HIDDEN_DIM), bf16),
            pltpu.VMEM((BATCH, HIDDEN_DIM), f32),
            pltpu.VMEM((BATCH, HIDDEN_DIM), bf16),
            pltpu.VMEM((BATCH, HIDDEN_DIM), f32),
        ],
        interpret=interpret,
    )(*args)
    return out, aux[:, 0]


def kernel(urls, emb_table, W_ih_f, W_hh_f, b_f, W_ih_b, W_hh_b, b_b,
           fc_w, fc_b, fc1_w, fc1_b, fc2_w, fc2_b, fc3_w, fc3_b):
    return _run(urls, emb_table, W_ih_f, W_hh_f, b_f, W_ih_b, W_hh_b, b_b,
                fc_w, fc_b, fc1_w, fc1_b, fc2_w, fc2_b, fc3_w, fc3_b)


# speedup vs baseline: 1.3768x; 1.0897x over previous
import functools

import jax
import jax.numpy as jnp
from jax.experimental import pallas as pl
from jax.experimental.pallas import tpu as pltpu

INPUT_DIM = 256
EMB_DIM = 128
HIDDEN_DIM = 512
BATCH = 128
SEQ = 200
H4 = 4 * HIDDEN_DIM
CHUNK = 8


def _lstm_kernel(urls_ref, emb_ref, wihf_ref, whhf_ref, bf_ref,
                 wihb_ref, whhb_ref, bb_ref,
                 fcw_ref, fcb_ref, fc1w_ref, fc1b_ref,
                 fc2w_ref, fc2b_ref, fc3w_ref, fc3b_ref,
                 out_ref, aux_ref,
                 mf_scr, mb_scr, gif_scr, gib_scr):
    f32 = jnp.float32
    bf16 = jnp.bfloat16
    rows = CHUNK * BATCH

    emb = emb_ref[...]
    mf_scr[...] = (jnp.dot(emb, wihf_ref[...], preferred_element_type=f32)
                   + bf_ref[...]).astype(bf16)
    mb_scr[...] = (jnp.dot(emb, wihb_ref[...], preferred_element_type=f32)
                   + bb_ref[...]).astype(bf16)

    iota = jax.lax.broadcasted_iota(jnp.int32, (rows, INPUT_DIM), 1)

    whhf = whhf_ref[...]
    whhb = whhb_ref[...]
    mf = mf_scr[...]
    mb = mb_scr[...]

    def sig(x):
        return 0.5 * jnp.tanh(0.5 * x) + 0.5

    def step_dir(gin, h, c, whh):
        gates = jnp.dot(h, whh, preferred_element_type=f32) + gin.astype(f32)
        i = sig(gates[:, 0:HIDDEN_DIM])
        f = sig(gates[:, HIDDEN_DIM:2 * HIDDEN_DIM])
        g = jnp.tanh(gates[:, 2 * HIDDEN_DIM:3 * HIDDEN_DIM])
        o = sig(gates[:, 3 * HIDDEN_DIM:])
        c = f * c + i * g
        h = (o * jnp.tanh(c)).astype(bf16)
        return h, c

    def chunk_body(k, carry):
        ids_f = urls_ref[pl.ds(k * rows, rows), :]
        ids_b = urls_ref[pl.ds((SEQ * BATCH) - (k + 1) * rows, rows), :]
        gif_scr[...] = jnp.dot((ids_f == iota).astype(bf16), mf,
                               preferred_element_type=f32).astype(bf16)
        gib_scr[...] = jnp.dot((ids_b == iota).astype(bf16), mb,
                               preferred_element_type=f32).astype(bf16)

        def one_step(j, carry):
            hf, cf, hb, cb = carry
            gf = gif_scr[pl.ds(j * BATCH, BATCH), :]
            gb = gib_scr[pl.ds((CHUNK - 1 - j) * BATCH, BATCH), :]
            hf, cf = step_dir(gf, hf, cf, whhf)
            hb, cb = step_dir(gb, hb, cb, whhb)
            return hf, cf, hb, cb

        def body2(j2, carry):
            carry = one_step(2 * j2, carry)
            carry = one_step(2 * j2 + 1, carry)
            return carry

        return jax.lax.fori_loop(0, CHUNK // 2, body2, carry)

    hf = jnp.zeros((BATCH, HIDDEN_DIM), bf16)
    hb = jnp.zeros((BATCH, HIDDEN_DIM), bf16)
    cf = jnp.zeros((BATCH, HIDDEN_DIM), f32)
    cb = jnp.zeros((BATCH, HIDDEN_DIM), f32)
    hf, cf, hb, cb = jax.lax.fori_loop(0, SEQ // CHUNK, chunk_body,
                                       (hf, cf, hb, cb))

    hidden = jnp.concatenate([hf, hb], axis=1)
    aux_ref[...] = jnp.dot(hidden, fcw_ref[...], preferred_element_type=f32) + fcb_ref[...]
    o1 = (jnp.dot(hidden, fc1w_ref[...], preferred_element_type=f32)
          + fc1b_ref[...]).astype(bf16)
    o2 = (jnp.dot(o1, fc2w_ref[...], preferred_element_type=f32)
          + fc2b_ref[...]).astype(bf16)
    out_ref[...] = jnp.dot(o2, fc3w_ref[...], preferred_element_type=f32) + fc3b_ref[...]


@functools.partial(jax.jit, static_argnames=("interpret",))
def _run(urls, emb_table, W_ih_f, W_hh_f, b_f, W_ih_b, W_hh_b, b_b,
         fc_w, fc_b, fc1_w, fc1_b, fc2_w, fc2_b, fc3_w, fc3_b,
         interpret=False):
    urls_flat = urls.T.reshape(SEQ * BATCH, 1).astype(jnp.int32)
    f32 = jnp.float32
    bf16 = jnp.bfloat16
    args = (
        urls_flat,
        emb_table,
        W_ih_f.T, W_hh_f.T.astype(bf16), b_f.reshape(1, H4),
        W_ih_b.T, W_hh_b.T.astype(bf16), b_b.reshape(1, H4),
        fc_w.T.astype(bf16), fc_b.reshape(1, 1),
        fc1_w.T.astype(bf16), fc1_b.reshape(1, H4),
        fc2_w.T.astype(bf16), fc2_b.reshape(1, 2 * HIDDEN_DIM),
        fc3_w.T.astype(bf16), fc3_b.reshape(1, 2),
    )
    out, aux = pl.pallas_call(
        _lstm_kernel,
        out_shape=(
            jax.ShapeDtypeStruct((BATCH, 2), f32),
            jax.ShapeDtypeStruct((BATCH, 1), f32),
        ),
        scratch_shapes=[
            pltpu.VMEM((INPUT_DIM, H4), bf16),
            pltpu.VMEM((INPUT_DIM, H4), bf16),
            pltpu.VMEM((CHUNK * BATCH, H4), bf16),
            pltpu.VMEM((CHUNK * BATCH, H4), bf16),
        ],
        interpret=interpret,
    )(*args)
    return out, aux[:, 0]


def kernel(urls, emb_table, W_ih_f, W_hh_f, b_f, W_ih_b, W_hh_b, b_b,
           fc_w, fc_b, fc1_w, fc1_b, fc2_w, fc2_b, fc3_w, fc3_b):
    return _run(urls, emb_table, W_ih_f, W_hh_f, b_f, W_ih_b, W_hh_b, b_b,
                fc_w, fc_b, fc1_w, fc1_b, fc2_w, fc2_b, fc3_w, fc3_b)


# speedup vs baseline: 1.7054x; 1.2387x over previous
import functools

import jax
import jax.numpy as jnp
from jax.experimental import pallas as pl
from jax.experimental.pallas import tpu as pltpu

INPUT_DIM = 256
EMB_DIM = 128
HIDDEN_DIM = 512
BATCH = 128
SEQ = 200
H4 = 4 * HIDDEN_DIM
CHUNK = 8


def _lstm_kernel(urls_ref, emb_ref, wihf_ref, whhf_ref, bf_ref,
                 wihb_ref, whhb_ref, bb_ref,
                 fcw_ref, fcb_ref, fc1w_ref, fc1b_ref,
                 fc2w_ref, fc2b_ref, fc3w_ref, fc3b_ref,
                 out_ref, aux_ref,
                 mf_scr, mb_scr, gif_scr, gib_scr):
    f32 = jnp.float32
    bf16 = jnp.bfloat16
    rows = CHUNK * BATCH

    emb = emb_ref[...]
    mf_scr[...] = (jnp.dot(emb, wihf_ref[...], preferred_element_type=f32)
                   + bf_ref[...]).astype(bf16)
    mb_scr[...] = (jnp.dot(emb, wihb_ref[...], preferred_element_type=f32)
                   + bb_ref[...]).astype(bf16)

    iota = jax.lax.broadcasted_iota(jnp.int32, (rows, INPUT_DIM), 1)

    whhf = whhf_ref[...]
    whhb = whhb_ref[...]
    mf = mf_scr[...]
    mb = mb_scr[...]

    def sig(x):
        return 0.5 * jnp.tanh(0.5 * x) + 0.5

    def step_dir(gin, h, c, whh):
        gates = jnp.dot(h, whh, preferred_element_type=f32) + gin.astype(f32)
        i = sig(gates[:, 0:HIDDEN_DIM])
        f = sig(gates[:, HIDDEN_DIM:2 * HIDDEN_DIM])
        g = jnp.tanh(gates[:, 2 * HIDDEN_DIM:3 * HIDDEN_DIM])
        o = sig(gates[:, 3 * HIDDEN_DIM:])
        c = f * c + i * g
        h = (o * jnp.tanh(c)).astype(bf16)
        return h, c

    def chunk_body(k, carry):
        ids_f = urls_ref[pl.ds(k * rows, rows), :]
        ids_b = urls_ref[pl.ds((SEQ * BATCH) - (k + 1) * rows, rows), :]
        gif_scr[...] = jnp.dot((ids_f == iota).astype(bf16), mf,
                               preferred_element_type=f32).astype(bf16)
        gib_scr[...] = jnp.dot((ids_b == iota).astype(bf16), mb,
                               preferred_element_type=f32).astype(bf16)

        hf, cf, hb, cb = carry
        for j in range(CHUNK):
            gf = gif_scr[j * BATCH:(j + 1) * BATCH, :]
            gb = gib_scr[(CHUNK - 1 - j) * BATCH:(CHUNK - j) * BATCH, :]
            hf, cf = step_dir(gf, hf, cf, whhf)
            hb, cb = step_dir(gb, hb, cb, whhb)
        return hf, cf, hb, cb

    hf = jnp.zeros((BATCH, HIDDEN_DIM), bf16)
    hb = jnp.zeros((BATCH, HIDDEN_DIM), bf16)
    cf = jnp.zeros((BATCH, HIDDEN_DIM), f32)
    cb = jnp.zeros((BATCH, HIDDEN_DIM), f32)
    hf, cf, hb, cb = jax.lax.fori_loop(0, SEQ // CHUNK, chunk_body,
                                       (hf, cf, hb, cb))

    hidden = jnp.concatenate([hf, hb], axis=1)
    aux_ref[...] = jnp.dot(hidden, fcw_ref[...], preferred_element_type=f32) + fcb_ref[...]
    o1 = (jnp.dot(hidden, fc1w_ref[...], preferred_element_type=f32)
          + fc1b_ref[...]).astype(bf16)
    o2 = (jnp.dot(o1, fc2w_ref[...], preferred_element_type=f32)
          + fc2b_ref[...]).astype(bf16)
    out_ref[...] = jnp.dot(o2, fc3w_ref[...], preferred_element_type=f32) + fc3b_ref[...]


@functools.partial(jax.jit, static_argnames=("interpret",))
def _run(urls, emb_table, W_ih_f, W_hh_f, b_f, W_ih_b, W_hh_b, b_b,
         fc_w, fc_b, fc1_w, fc1_b, fc2_w, fc2_b, fc3_w, fc3_b,
         interpret=False):
    urls_flat = urls.T.reshape(SEQ * BATCH, 1).astype(jnp.int32)
    f32 = jnp.float32
    bf16 = jnp.bfloat16
    args = (
        urls_flat,
        emb_table,
        W_ih_f.T, W_hh_f.T.astype(bf16), b_f.reshape(1, H4),
        W_ih_b.T, W_hh_b.T.astype(bf16), b_b.reshape(1, H4),
        fc_w.T.astype(bf16), fc_b.reshape(1, 1),
        fc1_w.T.astype(bf16), fc1_b.reshape(1, H4),
        fc2_w.T.astype(bf16), fc2_b.reshape(1, 2 * HIDDEN_DIM),
        fc3_w.T.astype(bf16), fc3_b.reshape(1, 2),
    )
    out, aux = pl.pallas_call(
        _lstm_kernel,
        out_shape=(
            jax.ShapeDtypeStruct((BATCH, 2), f32),
            jax.ShapeDtypeStruct((BATCH, 1), f32),
        ),
        scratch_shapes=[
            pltpu.VMEM((INPUT_DIM, H4), bf16),
            pltpu.VMEM((INPUT_DIM, H4), bf16),
            pltpu.VMEM((CHUNK * BATCH, H4), bf16),
            pltpu.VMEM((CHUNK * BATCH, H4), bf16),
        ],
        interpret=interpret,
    )(*args)
    return out, aux[:, 0]


def kernel(urls, emb_table, W_ih_f, W_hh_f, b_f, W_ih_b, W_hh_b, b_b,
           fc_w, fc_b, fc1_w, fc1_b, fc2_w, fc2_b, fc3_w, fc3_b):
    return _run(urls, emb_table, W_ih_f, W_hh_f, b_f, W_ih_b, W_hh_b, b_b,
                fc_w, fc_b, fc1_w, fc1_b, fc2_w, fc2_b, fc3_w, fc3_b)
